# Initial kernel scaffold; baseline (speedup 1.0000x reference)
import jax
import jax.numpy as jnp
from jax.experimental import pallas as pl


def kernel(x, cat_ids, W, b):
    raise NotImplementedError("write your pallas kernel here")


# speedup vs baseline: 3.7019x; 3.7019x over previous
import jax
import jax.numpy as jnp
from jax.experimental import pallas as pl


def _body(ids_ref, x_ref, w_ref, b_ref, o_ref):
    c = pl.program_id(0)
    mask = ids_ref[...] == c
    xm = jnp.where(mask, x_ref[...], 0.0)
    acc = jnp.dot(xm, w_ref[0], preferred_element_type=jnp.float32)
    acc = acc + mask.astype(jnp.float32) * b_ref[0]

    @pl.when(c == 0)
    def _init():
        o_ref[...] = acc

    @pl.when(c != 0)
    def _accum():
        o_ref[...] += acc


def kernel(x, cat_ids, W, b):
    N, IN = x.shape
    C, _, OUT = W.shape
    ids2 = cat_ids.astype(jnp.int32).reshape(N, 1)
    b3 = b.reshape(C, 1, OUT)
    return pl.pallas_call(
        _body,
        grid=(C,),
        in_specs=[
            pl.BlockSpec((N, 1), lambda c: (0, 0)),
            pl.BlockSpec((N, IN), lambda c: (0, 0)),
            pl.BlockSpec((1, IN, OUT), lambda c: (c, 0, 0)),
            pl.BlockSpec((1, 1, OUT), lambda c: (c, 0, 0)),
        ],
        out_specs=pl.BlockSpec((N, OUT), lambda c: (0, 0)),
        out_shape=jax.ShapeDtypeStruct((N, OUT), jnp.float32),
    )(ids2, x, W, b3)
